# baseline (speedup 1.0000x reference)
import functools

import jax
import jax.numpy as jnp
from jax import lax
from jax.experimental import pallas as pl
from jax.experimental.pallas import tpu as pltpu
from jax.experimental.pallas import tpu_sc as plsc

N = 16384
D = 64
L = 16
NC = 2
NS = 16
NW = NC * NS
R = N // NW
CH = 128
NCHUNK = R // CH

_mesh = plsc.VectorSubcoreMesh(core_axis_name="c", subcore_axis_name="s")


@functools.partial(
    pl.kernel,
    out_type=jax.ShapeDtypeStruct((N,), jnp.float32),
    mesh=_mesh,
    compiler_params=pltpu.CompilerParams(needs_layout_passes=False),
    scratch_types=[
        pltpu.VMEM((2, CH, D), jnp.float32),
        pltpu.VMEM((R,), jnp.float32),
        pltpu.VMEM((1, D), jnp.float32),
        pltpu.SemaphoreType.DMA,
        pltpu.SemaphoreType.DMA,
    ],
)
def _sc_matvec(user_hbm, items_hbm, out_hbm, buf, out_v, u_v, sem_in, sem_u):
    wid = lax.axis_index("s") * NC + lax.axis_index("c")
    base = wid * R

    ucp = pltpu.async_copy(user_hbm, u_v, sem_u)
    copies = [
        pltpu.async_copy(items_hbm.at[pl.ds(base + c * CH, CH)],
                         buf.at[c % 2], sem_in)
        for c in range(2)
    ]
    ucp.wait()
    u = [u_v[0, pl.ds(c * L, L)] for c in range(D // L)]

    lane = lax.iota(jnp.int32, L)

    for ch in range(NCHUNK):
        copies[ch].wait()
        cur = ch % 2

        def group_body(g, _, cur=cur, off=ch * CH):
            vec = jnp.zeros((L,), jnp.float32)
            for r in range(L):
                i = g * L + r
                s = buf[cur, i, pl.ds(0, L)] * u[0]
                for c in range(1, D // L):
                    s = s + buf[cur, i, pl.ds(c * L, L)] * u[c]
                vec = jnp.where(lane == r, jnp.sum(s), vec)
            out_v[pl.ds(off + g * L, L)] = vec
            return 0

        lax.fori_loop(0, CH // L, group_body, 0)

        if ch + 2 < NCHUNK:
            copies.append(
                pltpu.async_copy(items_hbm.at[pl.ds(base + (ch + 2) * CH, CH)],
                                 buf.at[ch % 2], sem_in))

    pltpu.sync_copy(out_v, out_hbm.at[pl.ds(base, R)])


def kernel(user_emb, items_emb):
    return _sc_matvec(user_emb, items_emb)
